# Initial kernel scaffold; baseline (speedup 1.0000x reference)
import jax
import jax.numpy as jnp
from jax.experimental import pallas as pl

N = 256


def _lu_batch(A):
    B = A.shape[0]
    rows = jax.lax.broadcasted_iota(jnp.int32, (1, N, 1), 1)
    cols = jax.lax.broadcasted_iota(jnp.int32, (1, 1, N), 2)

    def body(k, A):
        row = jax.lax.dynamic_slice(A, (0, k, 0), (B, 1, N))
        piv = jnp.sum(jnp.where(cols == k, row, 0.0), axis=2,
                      keepdims=True)
        colmask = (cols == k).astype(A.dtype)
        col = jnp.sum(A * colmask, axis=2, keepdims=True)
        c = jnp.where(rows > k, col / piv, 0.0)
        rp = jnp.where(cols > k, row, 0.0) + (piv - 1.0) * colmask
        return A - c * rp

    return jax.lax.fori_loop(0, N, body, A)


def _place(vals_rc):
    rows = jax.lax.broadcasted_iota(jnp.int32, (N, N), 0)
    cols = jax.lax.broadcasted_iota(jnp.int32, (N, N), 1)
    out = jnp.zeros((N, N), jnp.float32)
    for (r, c), v in vals_rc:
        out = out + v * ((rows == r) & (cols == c)).astype(jnp.float32)
    return out


def _lu_kernel(x_ref, o_ref):
    A0 = jnp.stack([x_ref[0], x_ref[1], x_ref[2], x_ref[5], x_ref[6]])
    L0 = _lu_batch(A0)
    b0, b1, b2, b5, b6 = L0[0], L0[1], L0[2], L0[3], L0[4]

    a3 = x_ref[3] - _place([
        ((0, 0), b1[1, 1] + b2[2, 2]),
        ((0, 1), b2[2, 3]),
        ((1, 0), b2[3, 2]),
        ((1, 1), b2[3, 3]),
    ])
    a7 = x_ref[7] - _place([
        ((0, 0), b5[1, 1] + b6[3, 3]),
        ((0, 1), b6[3, 4]),
        ((1, 0), b6[4, 3]),
        ((1, 1), b6[4, 4]),
    ])

    L1 = _lu_batch(jnp.stack([a3, a7]))
    b3, b7 = L1[0], L1[1]

    a8 = x_ref[8] - _place([((0, 0), b0[1, 1] + b3[1, 1] + b7[1, 1])])
    b8 = _lu_batch(a8[None])[0]

    o_ref[0] = b0
    o_ref[1] = b1
    o_ref[2] = b2
    o_ref[3] = b3
    o_ref[4] = x_ref[4]
    o_ref[5] = b5
    o_ref[6] = b6
    o_ref[7] = b7
    o_ref[8] = b8


def kernel(input):
    return pl.pallas_call(
        _lu_kernel,
        out_shape=jax.ShapeDtypeStruct((9, N, N), jnp.float32),
    )(input)


# speedup vs baseline: 144.3490x; 144.3490x over previous
import jax
import jax.numpy as jnp
from jax.experimental import pallas as pl
from jax.experimental.pallas import tpu as pltpu

N = 256


def _lu_ref(sref, B):
    rows = jax.lax.broadcasted_iota(jnp.int32, (1, N, 1), 1)
    cols = jax.lax.broadcasted_iota(jnp.int32, (1, 1, N), 2)

    def body(k, carry):
        A = sref[0:B]
        row = sref[0:B, pl.ds(k, 1), :]
        colmask = (cols == k).astype(jnp.float32)
        piv = jnp.sum(row * colmask, axis=2, keepdims=True)
        col = jnp.sum(A * colmask, axis=2, keepdims=True)
        c = jnp.where(rows > k, col / piv, 0.0)
        rp = jnp.where(cols > k, row, 0.0) + (piv - 1.0) * colmask
        sref[0:B] = A - c * rp
        return carry

    jax.lax.fori_loop(0, N, body, 0)


def _masks_2x2():
    r = jax.lax.broadcasted_iota(jnp.int32, (N, N), 0)
    c = jax.lax.broadcasted_iota(jnp.int32, (N, N), 1)
    def m(i, j):
        return ((r == i) & (c == j)).astype(jnp.float32)
    return m


def _lu_kernel(x_ref, o_ref, s):
    m = _masks_2x2()

    s[0] = x_ref[0]
    s[1] = x_ref[1]
    s[2] = x_ref[2]
    s[3] = x_ref[5]
    s[4] = x_ref[6]
    _lu_ref(s, 5)
    o_ref[0] = s[0]
    o_ref[1] = s[1]
    o_ref[2] = s[2]
    o_ref[5] = s[3]
    o_ref[6] = s[4]
    o_ref[4] = x_ref[4]

    v8_b0 = s[0, 1:2, 1:2]

    b1, b2, b5, b6 = s[1], s[2], s[3], s[4]
    corr3 = ((b1[1:2, 1:2] + b2[2:3, 2:3]) * m(0, 0)
             + b2[2:3, 3:4] * m(0, 1)
             + b2[3:4, 2:3] * m(1, 0)
             + b2[3:4, 3:4] * m(1, 1))
    corr7 = ((b5[1:2, 1:2] + b6[3:4, 3:4]) * m(0, 0)
             + b6[3:4, 4:5] * m(0, 1)
             + b6[4:5, 3:4] * m(1, 0)
             + b6[4:5, 4:5] * m(1, 1))

    s[0] = x_ref[3] - corr3
    s[1] = x_ref[7] - corr7
    _lu_ref(s, 2)
    o_ref[3] = s[0]
    o_ref[7] = s[1]

    corr8 = (v8_b0 + s[0, 1:2, 1:2] + s[1, 1:2, 1:2]) * m(0, 0)
    s[0] = x_ref[8] - corr8
    _lu_ref(s, 1)
    o_ref[8] = s[0]


def kernel(input):
    return pl.pallas_call(
        _lu_kernel,
        out_shape=jax.ShapeDtypeStruct((9, N, N), jnp.float32),
        scratch_shapes=[pltpu.VMEM((5, N, N), jnp.float32)],
    )(input)


# speedup vs baseline: 182.3105x; 1.2630x over previous
import jax
import jax.numpy as jnp
from jax.experimental import pallas as pl
from jax.experimental.pallas import tpu as pltpu

N = 256


ROW_W = 64
COL_W = 128


def _lu_ref(sref, B):
    for ch in range(N // ROW_W):
        roff = ch * ROW_W
        coff = (roff // COL_W) * COL_W
        MR, MC = N - roff, N - coff
        rows = jax.lax.broadcasted_iota(jnp.int32, (1, MR, 1), 1)
        cols = jax.lax.broadcasted_iota(jnp.int32, (1, 1, MC), 2)

        def body(k, carry, roff=roff, coff=coff, MR=MR, MC=MC,
                 rows=rows, cols=cols):
            kl_r = k + (roff - roff)
            kl_c = k + (roff - coff)
            A = sref[0:B, roff:, coff:]
            row = sref[0:B, pl.ds(roff + k, 1), :][:, :, coff:]
            colmask = (cols == kl_c).astype(jnp.float32)
            piv = jnp.sum(row * colmask, axis=2, keepdims=True)
            col = jnp.sum(A * colmask, axis=2, keepdims=True)
            c = jnp.where(rows > kl_r, col / piv, 0.0)
            rp = jnp.where(cols > kl_c, row, 0.0) + (piv - 1.0) * colmask
            sref[0:B, roff:, coff:] = A - c * rp
            return carry

        jax.lax.fori_loop(0, ROW_W, body, 0)


def _masks_2x2():
    r = jax.lax.broadcasted_iota(jnp.int32, (N, N), 0)
    c = jax.lax.broadcasted_iota(jnp.int32, (N, N), 1)
    def m(i, j):
        return ((r == i) & (c == j)).astype(jnp.float32)
    return m


def _lu_kernel(x_ref, o_ref, s):
    m = _masks_2x2()

    s[0] = x_ref[0]
    s[1] = x_ref[1]
    s[2] = x_ref[2]
    s[3] = x_ref[5]
    s[4] = x_ref[6]
    _lu_ref(s, 5)
    o_ref[0] = s[0]
    o_ref[1] = s[1]
    o_ref[2] = s[2]
    o_ref[5] = s[3]
    o_ref[6] = s[4]
    o_ref[4] = x_ref[4]

    v8_b0 = s[0, 1:2, 1:2]

    b1, b2, b5, b6 = s[1], s[2], s[3], s[4]
    corr3 = ((b1[1:2, 1:2] + b2[2:3, 2:3]) * m(0, 0)
             + b2[2:3, 3:4] * m(0, 1)
             + b2[3:4, 2:3] * m(1, 0)
             + b2[3:4, 3:4] * m(1, 1))
    corr7 = ((b5[1:2, 1:2] + b6[3:4, 3:4]) * m(0, 0)
             + b6[3:4, 4:5] * m(0, 1)
             + b6[4:5, 3:4] * m(1, 0)
             + b6[4:5, 4:5] * m(1, 1))

    s[0] = x_ref[3] - corr3
    s[1] = x_ref[7] - corr7
    _lu_ref(s, 2)
    o_ref[3] = s[0]
    o_ref[7] = s[1]

    corr8 = (v8_b0 + s[0, 1:2, 1:2] + s[1, 1:2, 1:2]) * m(0, 0)
    s[0] = x_ref[8] - corr8
    _lu_ref(s, 1)
    o_ref[8] = s[0]


def kernel(input):
    return pl.pallas_call(
        _lu_kernel,
        out_shape=jax.ShapeDtypeStruct((9, N, N), jnp.float32),
        scratch_shapes=[pltpu.VMEM((5, N, N), jnp.float32)],
    )(input)


# speedup vs baseline: 184.3302x; 1.0111x over previous
import jax
import jax.numpy as jnp
from jax.experimental import pallas as pl
from jax.experimental.pallas import tpu as pltpu

N = 256


ROW_W = 64
COL_W = 128
R = 8


def _lu_ref(sref, B):
    for ch in range(N // ROW_W):
        roff = ch * ROW_W
        coff = (roff // COL_W) * COL_W
        MR, MC = N - roff, N - coff
        rows = jax.lax.broadcasted_iota(jnp.int32, (1, MR, 1), 1)
        cols = jax.lax.broadcasted_iota(jnp.int32, (1, 1, MC), 2)
        ecol = jax.lax.broadcasted_iota(jnp.int32, (MC, R), 0)
        eidx = jax.lax.broadcasted_iota(jnp.int32, (MC, R), 1)
        i8c = jax.lax.broadcasted_iota(jnp.int32, (1, 1, R), 2)
        i8r = jax.lax.broadcasted_iota(jnp.int32, (1, R, 1), 1)

        def body(t, carry, roff=roff, coff=coff, MR=MR, MC=MC,
                 rows=rows, cols=cols, ecol=ecol, eidx=eidx,
                 i8c=i8c, i8r=i8r):
            kr = t * R
            kc = (roff - coff) + t * R
            A = sref[0:B, roff:, coff:]
            Rp = sref[0:B, pl.ds(roff + t * R, R), :][:, :, coff:]
            E = (ecol == eidx + kc).astype(jnp.float32)
            P = jax.lax.dot_general(A, E, (((2,), (0,)), ((), ())))
            S = jax.lax.dot_general(Rp, E, (((2,), (0,)), ((), ())))

            cs, rps = [], []
            for j in range(R):
                piv = S[:, j:j + 1, j:j + 1]
                rowj = Rp[:, j:j + 1, :]
                colj = P[:, :, j:j + 1]
                scol = S[:, :, j:j + 1]
                cmask = (cols == kc + j).astype(jnp.float32)
                c = jnp.where(rows > kr + j, colj / piv, 0.0)
                cpan = jnp.where(i8r > j, scol / piv, 0.0)
                rp = jnp.where(cols > kc + j, rowj, 0.0) + (piv - 1.0) * cmask
                rppan = (jnp.where(i8c > j, S[:, j:j + 1, :], 0.0)
                         + (piv - 1.0) * (i8c == j).astype(jnp.float32))
                Rp = Rp - cpan * rp
                P = P - c * rppan
                S = S - cpan * rppan
                cs.append(c)
                rps.append(rp)

            C = jnp.concatenate(cs, axis=2)
            Rm = jnp.concatenate(rps, axis=1)
            upd = jax.lax.dot_general(C, Rm, (((2,), (1,)), ((0,), (0,))))
            sref[0:B, roff:, coff:] = A - upd
            return carry

        jax.lax.fori_loop(0, ROW_W // R, body, 0)


def _masks_2x2():
    r = jax.lax.broadcasted_iota(jnp.int32, (N, N), 0)
    c = jax.lax.broadcasted_iota(jnp.int32, (N, N), 1)
    def m(i, j):
        return ((r == i) & (c == j)).astype(jnp.float32)
    return m


def _lu_kernel(x_ref, o_ref, s):
    m = _masks_2x2()

    s[0] = x_ref[0]
    s[1] = x_ref[1]
    s[2] = x_ref[2]
    s[3] = x_ref[5]
    s[4] = x_ref[6]
    _lu_ref(s, 5)
    o_ref[0] = s[0]
    o_ref[1] = s[1]
    o_ref[2] = s[2]
    o_ref[5] = s[3]
    o_ref[6] = s[4]
    o_ref[4] = x_ref[4]

    v8_b0 = s[0, 1:2, 1:2]

    b1, b2, b5, b6 = s[1], s[2], s[3], s[4]
    corr3 = ((b1[1:2, 1:2] + b2[2:3, 2:3]) * m(0, 0)
             + b2[2:3, 3:4] * m(0, 1)
             + b2[3:4, 2:3] * m(1, 0)
             + b2[3:4, 3:4] * m(1, 1))
    corr7 = ((b5[1:2, 1:2] + b6[3:4, 3:4]) * m(0, 0)
             + b6[3:4, 4:5] * m(0, 1)
             + b6[4:5, 3:4] * m(1, 0)
             + b6[4:5, 4:5] * m(1, 1))

    s[0] = x_ref[3] - corr3
    s[1] = x_ref[7] - corr7
    _lu_ref(s, 2)
    o_ref[3] = s[0]
    o_ref[7] = s[1]

    corr8 = (v8_b0 + s[0, 1:2, 1:2] + s[1, 1:2, 1:2]) * m(0, 0)
    s[0] = x_ref[8] - corr8
    _lu_ref(s, 1)
    o_ref[8] = s[0]


def kernel(input):
    return pl.pallas_call(
        _lu_kernel,
        out_shape=jax.ShapeDtypeStruct((9, N, N), jnp.float32),
        scratch_shapes=[pltpu.VMEM((5, N, N), jnp.float32)],
    )(input)
